# Initial kernel scaffold; baseline (speedup 1.0000x reference)
import jax
import jax.numpy as jnp
from jax.experimental import pallas as pl


def kernel(x, wte, W, b):
    raise NotImplementedError("write your pallas kernel here")


# speedup vs baseline: 5.6944x; 5.6944x over previous
import functools

import jax
import jax.numpy as jnp
from jax import lax
from jax.experimental import pallas as pl
from jax.experimental.pallas import tpu as pltpu
from jax.experimental.pallas import tpu_sc as plsc

B, L, V, D = 16384, 200, 128, 8
N = B * L
NC, NS = 2, 16
NW = NC * NS
PER_W = N // NW
C = 4096
NCHUNK = PER_W // C
GRP = C // 16


def _fuse_body(wte_ref, w_ref, b_ref, out_ref):
    out_ref[...] = lax.dot_general(
        wte_ref[...], w_ref[...],
        dimension_numbers=(((1,), (1,)), ((), ())),
        preferred_element_type=jnp.float32,
    ) + b_ref[...]


_fuse_table = pl.pallas_call(
    _fuse_body,
    out_shape=jax.ShapeDtypeStruct((V, D), jnp.float32),
)


def _sc_body(ft_hbm, idx_hbm, out_hbm, tbl_v, idx_v, out_v, sem_in, sem_out):
    wid = lax.axis_index("s") * NC + lax.axis_index("c")
    base = wid * PER_W

    pltpu.sync_copy(ft_hbm, tbl_v)

    pltpu.async_copy(idx_hbm.at[pl.ds(base, C)], idx_v.at[pl.ds(0, C)], sem_in)

    iota16 = lax.iota(jnp.int32, 16)
    iota8x = iota16 * D

    @pl.loop(0, NCHUNK)
    def _chunk(c):
        slot = c % 2
        ioff = slot * C
        ooff = slot * C * D

        pltpu.make_async_copy(
            idx_hbm.at[pl.ds(base + c * C, C)],
            idx_v.at[pl.ds(ioff, C)], sem_in).wait()

        @pl.when(c + 1 < NCHUNK)
        def _():
            pltpu.async_copy(
                idx_hbm.at[pl.ds(base + (c + 1) * C, C)],
                idx_v.at[pl.ds((1 - slot) * C, C)], sem_in)

        @pl.when(c >= 2)
        def _():
            pltpu.make_async_copy(
                out_v.at[pl.ds(ooff, C * D)],
                out_hbm.at[pl.ds((base + (c - 2) * C) * D, C * D)],
                sem_out).wait()

        @pl.loop(0, GRP, unroll=4)
        def _grp(g):
            xv = idx_v[pl.ds(ioff + g * 16, 16)]
            gbase = xv * D
            sbase = iota8x + (ooff + g * 16 * D)
            for d in range(D):
                vals = plsc.load_gather(tbl_v, [gbase + d])
                plsc.store_scatter(out_v, [sbase + d], vals)

        pltpu.async_copy(
            out_v.at[pl.ds(ooff, C * D)],
            out_hbm.at[pl.ds((base + c * C) * D, C * D)], sem_out)

    for t in (NCHUNK - 2, NCHUNK - 1):
        pltpu.make_async_copy(
            out_v.at[pl.ds((t % 2) * C * D, C * D)],
            out_hbm.at[pl.ds((base + t * C) * D, C * D)], sem_out).wait()


_sc_gather = pl.kernel(
    _sc_body,
    out_type=jax.ShapeDtypeStruct((N * D,), jnp.float32),
    mesh=plsc.VectorSubcoreMesh(
        core_axis_name="c", subcore_axis_name="s",
        num_cores=NC, num_subcores=NS),
    compiler_params=pltpu.CompilerParams(needs_layout_passes=False),
    scratch_types=[
        pltpu.VMEM((V * D,), jnp.float32),
        pltpu.VMEM((2 * C,), jnp.int32),
        pltpu.VMEM((2 * C * D,), jnp.float32),
        pltpu.SemaphoreType.DMA,
        pltpu.SemaphoreType.DMA,
    ],
)


@jax.jit
def kernel(x, wte, W, b):
    ft = _fuse_table(wte, W, b.reshape(1, D))
    out = _sc_gather(ft.reshape(V * D), x.reshape(N).astype(jnp.int32))
    return out.reshape(B, L, D)
